# baseline (speedup 1.0000x reference)
import functools

import jax
import jax.numpy as jnp
from jax import lax
from jax.experimental import pallas as pl
from jax.experimental.pallas import tpu as pltpu
from jax.experimental.pallas import tpu_sc as plsc

N = 10000
E = 320000
IN_DIM = 128
D = 16
H = 8
HD = H * D

NC = 2
NS = 16
NW = NC * NS
EPW = E // NW
B = 80
NCHUNK = EPW // B
NPAD = 10240
RPS = NPAD // NS

_f32 = jnp.float32

_GDN = lax.GatherDimensionNumbers(offset_dims=(), collapsed_slice_dims=(0,),
                                  start_index_map=(0,))


def _vtake(vec, idx):
    return lax.gather(vec, idx[:, None], _GDN, (1,),
                      mode=lax.GatherScatterMode.PROMISE_IN_BOUNDS)


def _proj_body(h_ref, wq_ref, bq_ref, wk_ref, bk_ref, wv_ref, bv_ref,
               q_ref, k_ref, v_ref):
    hb = h_ref[...]
    q_ref[...] = (jnp.dot(hb, wq_ref[...], preferred_element_type=_f32)
                  + bq_ref[...]) * 0.25
    k_ref[...] = jnp.dot(hb, wk_ref[...], preferred_element_type=_f32) + bk_ref[...]
    v_ref[...] = jnp.dot(hb, wv_ref[...], preferred_element_type=_f32) + bv_ref[...]


def _proj_qkv(h, wq, bq, wk, bk, wv, bv):
    R = 2000
    grid = N // R
    mat = pl.BlockSpec((IN_DIM, HD), lambda i: (0, 0))
    vec = pl.BlockSpec((1, HD), lambda i: (0, 0))
    blk = pl.BlockSpec((R, IN_DIM), lambda i: (i, 0))
    return pl.pallas_call(
        _proj_body,
        grid=(grid,),
        in_specs=[blk, mat, vec, mat, vec, mat, vec],
        out_specs=[blk, blk, blk],
        out_shape=[jax.ShapeDtypeStruct((N, HD), _f32)] * 3,
    )(h, wq, bq.reshape(1, HD), wk, bk.reshape(1, HD), wv, bv.reshape(1, HD))


NB = 5


def _sc_g_body(src_hbm, dst_hbm, k_hbm, q_hbm, g_hbm,
               srcb, dstb, krows, qrows, grows2,
               sk0, sk1, sk2, sk3, sk4, sq0, sq1, sq2, sq3, sq4, so0, so1):
    semk = (sk0, sk1, sk2, sk3, sk4)
    semq = (sq0, sq1, sq2, sq3, sq4)
    semo = (so0, so1)
    wid = lax.axis_index("s") * NC + lax.axis_index("c")
    wbase = wid * EPW

    for b in range(NB):
        base = wbase + b * B
        pltpu.sync_copy(src_hbm.at[pl.ds(base, B)], srcb.at[b])
        pltpu.sync_copy(dst_hbm.at[pl.ds(base, B)], dstb.at[b])
        pltpu.async_copy(k_hbm.at[srcb.at[b]], krows.at[b], semk[b])
        pltpu.async_copy(q_hbm.at[dstb.at[b]], qrows.at[b], semq[b])

    @pl.loop(0, NCHUNK, step=NB)
    def _grp(g):
        for b in range(NB):
            sb = b & 1
            base = wbase + (g + b) * B
            pltpu.make_async_copy(k_hbm.at[srcb.at[b]], krows.at[b],
                                  semk[b]).wait()
            pltpu.make_async_copy(q_hbm.at[dstb.at[b]], qrows.at[b],
                                  semq[b]).wait()

            def _drain():
                pltpu.make_async_copy(grows2.at[sb], g_hbm.at[pl.ds(0, B)],
                                      semo[sb]).wait()
            if b >= 2:
                _drain()
            else:
                pl.when(g > 0)(_drain)

            @plsc.parallel_loop(0, B)
            def _edge(j):
                for h in range(H):
                    sl = pl.ds(h * D, D)
                    grows2[sb, j, sl] = krows[b, j, sl] * qrows[b, j, sl]

            pltpu.async_copy(grows2.at[sb], g_hbm.at[pl.ds(base, B)], semo[sb])

            @pl.when(g + b + NB < NCHUNK)
            def _pf():
                nb_ = wbase + (g + b + NB) * B
                pltpu.sync_copy(src_hbm.at[pl.ds(nb_, B)], srcb.at[b])
                pltpu.sync_copy(dst_hbm.at[pl.ds(nb_, B)], dstb.at[b])
                pltpu.async_copy(k_hbm.at[srcb.at[b]], krows.at[b], semk[b])
                pltpu.async_copy(q_hbm.at[dstb.at[b]], qrows.at[b], semq[b])

    for sb in range(2):
        pltpu.make_async_copy(grows2.at[sb], g_hbm.at[pl.ds(0, B)],
                              semo[sb]).wait()


def _sc_g(src, dst, k, q):
    mesh = plsc.VectorSubcoreMesh(core_axis_name="c", subcore_axis_name="s")
    fn = pl.kernel(
        _sc_g_body,
        out_type=jax.ShapeDtypeStruct((E, HD), _f32),
        mesh=mesh,
        scratch_types=[
            pltpu.VMEM((NB, B), jnp.int32),
            pltpu.VMEM((NB, B), jnp.int32),
            pltpu.VMEM((NB, B, HD), _f32),
            pltpu.VMEM((NB, B, HD), _f32),
            pltpu.VMEM((2, B, HD), _f32),
        ] + [pltpu.SemaphoreType.DMA] * 12,
    )
    return fn(src, dst, k, q)


def _eproj_body(e_ref, we_ref, be_ref, proj_ref):
    proj_ref[...] = (jnp.dot(e_ref[...], we_ref[...],
                             preferred_element_type=_f32) + be_ref[...])


def _eproj_tc(e, we, be):
    R = 2000
    blk = pl.BlockSpec((R, IN_DIM), lambda i: (i, 0))
    return pl.pallas_call(
        _eproj_body,
        grid=(E // R,),
        in_specs=[blk,
                  pl.BlockSpec((IN_DIM, HD), lambda i: (0, 0)),
                  pl.BlockSpec((1, HD), lambda i: (0, 0))],
        out_specs=blk,
        out_shape=jax.ShapeDtypeStruct((E, HD), _f32),
    )(e, we, be.reshape(1, HD))


def _edge_body(proj_ref, g_ref, eout_ref, s_ref):
    sc = g_ref[...] * proj_ref[...]
    eout_ref[...] = sc
    r = lax.broadcasted_iota(jnp.int32, (HD, H), 0) // D
    c = lax.broadcasted_iota(jnp.int32, (HD, H), 1)
    m = (r == c).astype(_f32)
    ssum = jnp.dot(sc, m, preferred_element_type=_f32)
    s_ref[...] = jnp.exp(jnp.clip(ssum, -5.0, 5.0))


def _edge_tc(proj, g):
    R = 2000
    grid = E // R
    blk = pl.BlockSpec((R, IN_DIM), lambda i: (i, 0))
    return pl.pallas_call(
        _edge_body,
        grid=(grid,),
        in_specs=[blk, blk],
        out_specs=[blk, pl.BlockSpec((R, H), lambda i: (i, 0))],
        out_shape=[jax.ShapeDtypeStruct((E, HD), _f32),
                   jax.ShapeDtypeStruct((E, H), _f32)],
    )(proj, g)


ZGRP = NPAD // 8
ZRPS = ZGRP // NS


def _sc_scatter_body(src_hbm, dst_hbm, v_hbm, s_hbm, ow_hbm, oz_hbm,
                     src_v, dst_v, zdiv_v, mprev, s_v, vrows, zrows,
                     wsh, zsh, sem0, sem1):
    cid = lax.axis_index("c")
    sid = lax.axis_index("s")
    wid = sid * NC + cid
    zero16 = jnp.zeros((D,), _f32)
    sems = (sem0, sem1)
    wbase = wid * EPW
    iota = lax.iota(jnp.int32, D)
    izero = jnp.zeros((D,), jnp.int32)

    @pl.loop(0, B)
    def _z(r):
        for h in range(H):
            zrows[r, pl.ds(h * D, D)] = zero16

    @pl.loop(0, RPS // B)
    def _zc(t):
        pltpu.sync_copy(zrows, wsh.at[pl.ds(sid * RPS + t * B, B)])

    pltpu.sync_copy(zrows, zsh.at[pl.ds(sid * ZRPS, ZRPS)])

    for c in range(B // D):
        mprev[pl.ds(c * D, D)] = izero
    plsc.subcore_barrier()

    def _fetch(slot, ci):
        base = wbase + ci * B
        pltpu.sync_copy(src_hbm.at[pl.ds(base, B)], src_v.at[slot])
        pltpu.sync_copy(dst_hbm.at[pl.ds(base, B)], dst_v.at[slot])
        pltpu.sync_copy(s_hbm.at[pl.ds(base * H, B * H)], s_v.at[slot])
        pltpu.async_copy(v_hbm.at[src_v.at[slot]], vrows.at[slot], sems[slot])

    def _consume(slot):
        pltpu.make_async_copy(v_hbm.at[src_v.at[slot]], vrows.at[slot],
                              sems[slot]).wait()

        for c in range(B // D):
            slc = pl.ds(c * D, D)
            zdiv_v[slc] = lax.shift_right_logical(dst_v[slot, slc], 3)

        @plsc.parallel_loop(0, B // D)
        def _grp(c):
            dvec = dst_v[slot, pl.ds(c * D, D)] & 7
            mpv = mprev[pl.ds(c * D, D)]
            for q in range(D // 2):
                sv = s_v[slot, pl.ds((c * 8 + q) * (2 * H), 2 * H)]
                for t in range(2):
                    o = q * 2 + t
                    j = c * D + o
                    for h in range(H):
                        bc = _vtake(sv, jnp.full((D,), t * H + h, jnp.int32))
                        sl = pl.ds(h * D, D)
                        vrows[slot, j, sl] = vrows[slot, j, sl] * bc
                    zg = _vtake(sv, t * H + (iota & 7))
                    zvec = jnp.where(iota < 8, zg, 0.0)
                    zrows[j, pl.ds(mpv[o] * D, D)] = zero16
                    zrows[j, pl.ds(dvec[o] * D, D)] = zvec
            mprev[pl.ds(c * D, D)] = dvec

        pltpu.sync_copy(vrows.at[slot], wsh.at[dst_v.at[slot]], add=True)
        pltpu.sync_copy(zrows, zsh.at[zdiv_v], add=True)

    _fetch(0, 0)
    _fetch(1, 1)

    @pl.loop(0, NCHUNK // 2)
    def _super(i):
        c0 = i * 2
        _consume(0)
        _fetch(0, c0 + 2)
        _consume(1)

        @pl.when(c0 + 3 < NCHUNK)
        def _pf1():
            _fetch(1, c0 + 3)

    _consume(0)
    plsc.subcore_barrier()

    @pl.loop(0, RPS // B)
    def _out(t):
        row0 = sid * RPS + t * B
        pltpu.sync_copy(wsh.at[pl.ds(row0, B)], vrows.at[0])
        pltpu.sync_copy(vrows.at[0], ow_hbm.at[cid, pl.ds(row0, B)])

    pltpu.sync_copy(zsh.at[pl.ds(sid * ZRPS, ZRPS)], zrows)
    pltpu.sync_copy(zrows, oz_hbm.at[cid, pl.ds(sid * ZRPS, ZRPS)])


def _sc_scatter(src, dst, v, s_flat):
    mesh = plsc.VectorSubcoreMesh(core_axis_name="c", subcore_axis_name="s")
    fn = pl.kernel(
        _sc_scatter_body,
        out_type=(jax.ShapeDtypeStruct((NC, NPAD, HD), _f32),
                  jax.ShapeDtypeStruct((NC, ZGRP, HD), _f32)),
        mesh=mesh,
        scratch_types=[
            pltpu.VMEM((2, B), jnp.int32),
            pltpu.VMEM((2, B), jnp.int32),
            pltpu.VMEM((B,), jnp.int32),
            pltpu.VMEM((B,), jnp.int32),
            pltpu.VMEM((2, B * H), _f32),
            pltpu.VMEM((2, B, HD), _f32),
            pltpu.VMEM((B, HD), _f32),
            pltpu.VMEM_SHARED((NPAD, HD), _f32),
            pltpu.VMEM_SHARED((ZGRP, HD), _f32),
            pltpu.SemaphoreType.DMA,
            pltpu.SemaphoreType.DMA,
        ],
    )
    return fn(src, dst, v, s_flat)


def _combine_body(w_ref, z_ref, o_ref):
    w = w_ref[0] + w_ref[1]
    z = z_ref[0] + z_ref[1]
    j = lax.broadcasted_iota(jnp.int32, (D, HD), 0)
    l = lax.broadcasted_iota(jnp.int32, (D, HD), 1) // D
    m = (j == l).astype(_f32)
    zb = jnp.dot(z, m, preferred_element_type=_f32)
    o_ref[...] = w / (zb + 1e-6)


def _combine(ow, oz):
    R = 2048
    grid = NPAD // R
    return pl.pallas_call(
        _combine_body,
        grid=(grid,),
        in_specs=[pl.BlockSpec((NC, R, HD), lambda i: (0, i, 0)),
                  pl.BlockSpec((NC, R, D), lambda i: (0, i, 0))],
        out_specs=pl.BlockSpec((R, HD), lambda i: (i, 0)),
        out_shape=jax.ShapeDtypeStruct((NPAD, HD), _f32),
    )(ow, oz)


def kernel(h, e, edge_index, W_Q, b_Q, W_K, b_K, W_V, b_V, W_E, b_E):
    src = edge_index[0]
    dst = edge_index[1]
    q, k, v = _proj_qkv(h, W_Q, b_Q, W_K, b_K, W_V, b_V)
    proj = _eproj_tc(e, W_E, b_E)
    g = _sc_g(src, dst, k, q)
    e_out, s = _edge_tc(proj, g)
    ow, ozp = _sc_scatter(src, dst, v, s.reshape(E * H))
    h_out = _combine(ow, ozp.reshape(NC, NPAD, D))[:N]
    return h_out.reshape(N, H, D), e_out.reshape(E, H, D)


# speedup vs baseline: 1.0463x; 1.0463x over previous
import functools

import jax
import jax.numpy as jnp
from jax import lax
from jax.experimental import pallas as pl
from jax.experimental.pallas import tpu as pltpu
from jax.experimental.pallas import tpu_sc as plsc

N = 10000
E = 320000
IN_DIM = 128
D = 16
H = 8
HD = H * D

NC = 2
NS = 16
NW = NC * NS
EPW = E // NW
B = 80
NCHUNK = EPW // B
NPAD = 10240
RPS = NPAD // NS

_f32 = jnp.float32

_GDN = lax.GatherDimensionNumbers(offset_dims=(), collapsed_slice_dims=(0,),
                                  start_index_map=(0,))


def _vtake(vec, idx):
    return lax.gather(vec, idx[:, None], _GDN, (1,),
                      mode=lax.GatherScatterMode.PROMISE_IN_BOUNDS)


def _proj_body(h_ref, wq_ref, bq_ref, wk_ref, bk_ref, wv_ref, bv_ref,
               q_ref, k_ref, v_ref):
    hb = h_ref[...]
    q_ref[...] = (jnp.dot(hb, wq_ref[...], preferred_element_type=_f32)
                  + bq_ref[...]) * 0.25
    k_ref[...] = jnp.dot(hb, wk_ref[...], preferred_element_type=_f32) + bk_ref[...]
    v_ref[...] = jnp.dot(hb, wv_ref[...], preferred_element_type=_f32) + bv_ref[...]


def _proj_qkv(h, wq, bq, wk, bk, wv, bv):
    R = 2000
    grid = N // R
    mat = pl.BlockSpec((IN_DIM, HD), lambda i: (0, 0))
    vec = pl.BlockSpec((1, HD), lambda i: (0, 0))
    blk = pl.BlockSpec((R, IN_DIM), lambda i: (i, 0))
    return pl.pallas_call(
        _proj_body,
        grid=(grid,),
        in_specs=[blk, mat, vec, mat, vec, mat, vec],
        out_specs=[blk, blk, blk],
        out_shape=[jax.ShapeDtypeStruct((N, HD), _f32)] * 3,
    )(h, wq, bq.reshape(1, HD), wk, bk.reshape(1, HD), wv, bv.reshape(1, HD))


NB = 5


def _sc_g_body(src_hbm, dst_hbm, k_hbm, q_hbm, g_hbm,
               srcb, dstb, krows, qrows, grows2,
               sk0, sk1, sk2, sk3, sk4, sq0, sq1, sq2, sq3, sq4, so0, so1):
    semk = (sk0, sk1, sk2, sk3, sk4)
    semq = (sq0, sq1, sq2, sq3, sq4)
    semo = (so0, so1)
    wid = lax.axis_index("s") * NC + lax.axis_index("c")
    wbase = wid * EPW

    for b in range(NB):
        base = wbase + b * B
        pltpu.sync_copy(src_hbm.at[pl.ds(base, B)], srcb.at[b])
        pltpu.sync_copy(dst_hbm.at[pl.ds(base, B)], dstb.at[b])
        pltpu.async_copy(k_hbm.at[srcb.at[b]], krows.at[b], semk[b])
        pltpu.async_copy(q_hbm.at[dstb.at[b]], qrows.at[b], semq[b])

    @pl.loop(0, NCHUNK, step=NB)
    def _grp(g):
        for b in range(NB):
            sb = b & 1
            base = wbase + (g + b) * B
            pltpu.make_async_copy(k_hbm.at[srcb.at[b]], krows.at[b],
                                  semk[b]).wait()
            pltpu.make_async_copy(q_hbm.at[dstb.at[b]], qrows.at[b],
                                  semq[b]).wait()

            def _drain():
                pltpu.make_async_copy(grows2.at[sb], g_hbm.at[pl.ds(0, B)],
                                      semo[sb]).wait()
            if b >= 2:
                _drain()
            else:
                pl.when(g > 0)(_drain)

            @plsc.parallel_loop(0, B)
            def _edge(j):
                for h in range(H):
                    sl = pl.ds(h * D, D)
                    grows2[sb, j, sl] = krows[b, j, sl] * qrows[b, j, sl]

            pltpu.async_copy(grows2.at[sb], g_hbm.at[pl.ds(base, B)], semo[sb])

            @pl.when(g + b + NB < NCHUNK)
            def _pf():
                nb_ = wbase + (g + b + NB) * B
                pltpu.sync_copy(src_hbm.at[pl.ds(nb_, B)], srcb.at[b])
                pltpu.sync_copy(dst_hbm.at[pl.ds(nb_, B)], dstb.at[b])
                pltpu.async_copy(k_hbm.at[srcb.at[b]], krows.at[b], semk[b])
                pltpu.async_copy(q_hbm.at[dstb.at[b]], qrows.at[b], semq[b])

    for sb in range(2):
        pltpu.make_async_copy(grows2.at[sb], g_hbm.at[pl.ds(0, B)],
                              semo[sb]).wait()


def _sc_g(src, dst, k, q):
    mesh = plsc.VectorSubcoreMesh(core_axis_name="c", subcore_axis_name="s")
    fn = pl.kernel(
        _sc_g_body,
        out_type=jax.ShapeDtypeStruct((E, HD), _f32),
        mesh=mesh,
        scratch_types=[
            pltpu.VMEM((NB, B), jnp.int32),
            pltpu.VMEM((NB, B), jnp.int32),
            pltpu.VMEM((NB, B, HD), _f32),
            pltpu.VMEM((NB, B, HD), _f32),
            pltpu.VMEM((2, B, HD), _f32),
        ] + [pltpu.SemaphoreType.DMA] * 12,
    )
    return fn(src, dst, k, q)


def _edge_body(e_ref, we_ref, be_ref, g_ref, eout_ref, s_ref):
    proj = jnp.dot(e_ref[...], we_ref[...], preferred_element_type=_f32) + be_ref[...]
    sc = g_ref[...] * proj
    eout_ref[...] = sc
    r = lax.broadcasted_iota(jnp.int32, (HD, H), 0) // D
    c = lax.broadcasted_iota(jnp.int32, (HD, H), 1)
    m = (r == c).astype(_f32)
    ssum = jnp.dot(sc, m, preferred_element_type=_f32)
    s_ref[...] = jnp.exp(jnp.clip(ssum, -5.0, 5.0))


def _edge_tc(e, we, be, g):
    R = 2000
    grid = E // R
    blk = pl.BlockSpec((R, IN_DIM), lambda i: (i, 0))
    return pl.pallas_call(
        _edge_body,
        grid=(grid,),
        in_specs=[blk,
                  pl.BlockSpec((IN_DIM, HD), lambda i: (0, 0)),
                  pl.BlockSpec((1, HD), lambda i: (0, 0)),
                  blk],
        out_specs=[blk, pl.BlockSpec((R, H), lambda i: (i, 0))],
        out_shape=[jax.ShapeDtypeStruct((E, HD), _f32),
                   jax.ShapeDtypeStruct((E, H), _f32)],
    )(e, we, be.reshape(1, HD), g)


ZGRP = NPAD // 8
ZRPS = ZGRP // NS


def _sc_scatter_body(src_hbm, dst_hbm, v_hbm, s_hbm, ow_hbm, oz_hbm,
                     src_v, dst_v, zdiv_v, mprev, s_v, vrows, zrows,
                     wsh, zsh, sem0, sem1):
    cid = lax.axis_index("c")
    sid = lax.axis_index("s")
    wid = sid * NC + cid
    zero16 = jnp.zeros((D,), _f32)
    sems = (sem0, sem1)
    wbase = wid * EPW
    iota = lax.iota(jnp.int32, D)
    izero = jnp.zeros((D,), jnp.int32)

    @pl.loop(0, B)
    def _z(r):
        for h in range(H):
            zrows[r, pl.ds(h * D, D)] = zero16

    @pl.loop(0, RPS // B)
    def _zc(t):
        pltpu.sync_copy(zrows, wsh.at[pl.ds(sid * RPS + t * B, B)])

    pltpu.sync_copy(zrows, zsh.at[pl.ds(sid * ZRPS, ZRPS)])

    for c in range(B // D):
        mprev[pl.ds(c * D, D)] = izero
    plsc.subcore_barrier()

    def _fetch(slot, ci):
        base = wbase + ci * B
        pltpu.sync_copy(src_hbm.at[pl.ds(base, B)], src_v.at[slot])
        pltpu.sync_copy(dst_hbm.at[pl.ds(base, B)], dst_v.at[slot])
        pltpu.sync_copy(s_hbm.at[pl.ds(base * H, B * H)], s_v.at[slot])
        pltpu.async_copy(v_hbm.at[src_v.at[slot]], vrows.at[slot], sems[slot])

    def _consume(slot):
        pltpu.make_async_copy(v_hbm.at[src_v.at[slot]], vrows.at[slot],
                              sems[slot]).wait()

        for c in range(B // D):
            slc = pl.ds(c * D, D)
            zdiv_v[slc] = lax.shift_right_logical(dst_v[slot, slc], 3)

        @plsc.parallel_loop(0, B // D)
        def _grp(c):
            dvec = dst_v[slot, pl.ds(c * D, D)] & 7
            mpv = mprev[pl.ds(c * D, D)]
            for q in range(D // 2):
                sv = s_v[slot, pl.ds((c * 8 + q) * (2 * H), 2 * H)]
                for t in range(2):
                    o = q * 2 + t
                    j = c * D + o
                    for h in range(H):
                        bc = _vtake(sv, jnp.full((D,), t * H + h, jnp.int32))
                        sl = pl.ds(h * D, D)
                        vrows[slot, j, sl] = vrows[slot, j, sl] * bc
                    zg = _vtake(sv, t * H + (iota & 7))
                    zvec = jnp.where(iota < 8, zg, 0.0)
                    zrows[j, pl.ds(mpv[o] * D, D)] = zero16
                    zrows[j, pl.ds(dvec[o] * D, D)] = zvec
            mprev[pl.ds(c * D, D)] = dvec

        pltpu.sync_copy(vrows.at[slot], wsh.at[dst_v.at[slot]], add=True)
        pltpu.sync_copy(zrows, zsh.at[zdiv_v], add=True)

    _fetch(0, 0)
    _fetch(1, 1)

    @pl.loop(0, NCHUNK // 2)
    def _super(i):
        c0 = i * 2
        _consume(0)
        _fetch(0, c0 + 2)
        _consume(1)

        @pl.when(c0 + 3 < NCHUNK)
        def _pf1():
            _fetch(1, c0 + 3)

    _consume(0)
    plsc.subcore_barrier()

    @pl.loop(0, RPS // B)
    def _out(t):
        row0 = sid * RPS + t * B
        pltpu.sync_copy(wsh.at[pl.ds(row0, B)], vrows.at[0])
        pltpu.sync_copy(vrows.at[0], ow_hbm.at[cid, pl.ds(row0, B)])

    pltpu.sync_copy(zsh.at[pl.ds(sid * ZRPS, ZRPS)], zrows)
    pltpu.sync_copy(zrows, oz_hbm.at[cid, pl.ds(sid * ZRPS, ZRPS)])


def _sc_scatter(src, dst, v, s_flat):
    mesh = plsc.VectorSubcoreMesh(core_axis_name="c", subcore_axis_name="s")
    fn = pl.kernel(
        _sc_scatter_body,
        out_type=(jax.ShapeDtypeStruct((NC, NPAD, HD), _f32),
                  jax.ShapeDtypeStruct((NC, ZGRP, HD), _f32)),
        mesh=mesh,
        scratch_types=[
            pltpu.VMEM((2, B), jnp.int32),
            pltpu.VMEM((2, B), jnp.int32),
            pltpu.VMEM((B,), jnp.int32),
            pltpu.VMEM((B,), jnp.int32),
            pltpu.VMEM((2, B * H), _f32),
            pltpu.VMEM((2, B, HD), _f32),
            pltpu.VMEM((B, HD), _f32),
            pltpu.VMEM_SHARED((NPAD, HD), _f32),
            pltpu.VMEM_SHARED((ZGRP, HD), _f32),
            pltpu.SemaphoreType.DMA,
            pltpu.SemaphoreType.DMA,
        ],
    )
    return fn(src, dst, v, s_flat)


def _combine_body(w_ref, z_ref, o_ref):
    w = w_ref[0] + w_ref[1]
    z = z_ref[0] + z_ref[1]
    j = lax.broadcasted_iota(jnp.int32, (D, HD), 0)
    l = lax.broadcasted_iota(jnp.int32, (D, HD), 1) // D
    m = (j == l).astype(_f32)
    zb = jnp.dot(z, m, preferred_element_type=_f32)
    o_ref[...] = w / (zb + 1e-6)


def _combine(ow, oz):
    R = 2048
    grid = NPAD // R
    return pl.pallas_call(
        _combine_body,
        grid=(grid,),
        in_specs=[pl.BlockSpec((NC, R, HD), lambda i: (0, i, 0)),
                  pl.BlockSpec((NC, R, D), lambda i: (0, i, 0))],
        out_specs=pl.BlockSpec((R, HD), lambda i: (i, 0)),
        out_shape=jax.ShapeDtypeStruct((NPAD, HD), _f32),
    )(ow, oz)


def kernel(h, e, edge_index, W_Q, b_Q, W_K, b_K, W_V, b_V, W_E, b_E):
    src = edge_index[0]
    dst = edge_index[1]
    q, k, v = _proj_qkv(h, W_Q, b_Q, W_K, b_K, W_V, b_V)
    g = _sc_g(src, dst, k, q)
    e_out, s = _edge_tc(e, W_E, b_E, g)
    ow, ozp = _sc_scatter(src, dst, v, s.reshape(E * H))
    h_out = _combine(ow, ozp.reshape(NC, NPAD, D))[:N]
    return h_out.reshape(N, H, D), e_out.reshape(E, H, D)


# speedup vs baseline: 1.1993x; 1.1462x over previous
import functools

import jax
import jax.numpy as jnp
from jax import lax
from jax.experimental import pallas as pl
from jax.experimental.pallas import tpu as pltpu
from jax.experimental.pallas import tpu_sc as plsc

N = 10000
E = 320000
IN_DIM = 128
D = 16
H = 8
HD = H * D

NC = 2
NS = 16
NW = NC * NS
EPW = E // NW
B = 80
NCHUNK = EPW // B
NPAD = 10240
RPS = NPAD // NS

_f32 = jnp.float32

_GDN = lax.GatherDimensionNumbers(offset_dims=(), collapsed_slice_dims=(0,),
                                  start_index_map=(0,))


def _vtake(vec, idx):
    return lax.gather(vec, idx[:, None], _GDN, (1,),
                      mode=lax.GatherScatterMode.PROMISE_IN_BOUNDS)


def _proj_body(h_ref, wq_ref, bq_ref, wk_ref, bk_ref, wv_ref, bv_ref,
               q_ref, k_ref, v_ref):
    hb = h_ref[...]
    q_ref[...] = (jnp.dot(hb, wq_ref[...], preferred_element_type=_f32)
                  + bq_ref[...]) * 0.25
    k_ref[...] = jnp.dot(hb, wk_ref[...], preferred_element_type=_f32) + bk_ref[...]
    v_ref[...] = jnp.dot(hb, wv_ref[...], preferred_element_type=_f32) + bv_ref[...]


def _proj_qkv(h, wq, bq, wk, bk, wv, bv):
    R = 2000
    grid = N // R
    mat = pl.BlockSpec((IN_DIM, HD), lambda i: (0, 0))
    vec = pl.BlockSpec((1, HD), lambda i: (0, 0))
    blk = pl.BlockSpec((R, IN_DIM), lambda i: (i, 0))
    return pl.pallas_call(
        _proj_body,
        grid=(grid,),
        in_specs=[blk, mat, vec, mat, vec, mat, vec],
        out_specs=[blk, blk, blk],
        out_shape=[jax.ShapeDtypeStruct((N, HD), _f32)] * 3,
    )(h, wq, bq.reshape(1, HD), wk, bk.reshape(1, HD), wv, bv.reshape(1, HD))


NB = 5


def _sc_g_body(src_hbm, dst_hbm, k_hbm, q_hbm, g_hbm,
               srcb, dstb, krows, qrows, grows2,
               sk0, sk1, sk2, sk3, sk4, sq0, sq1, sq2, sq3, sq4, so0, so1):
    semk = (sk0, sk1, sk2, sk3, sk4)
    semq = (sq0, sq1, sq2, sq3, sq4)
    semo = (so0, so1)
    wid = lax.axis_index("s") * NC + lax.axis_index("c")
    wbase = wid * EPW

    for b in range(NB):
        base = wbase + b * B
        pltpu.sync_copy(src_hbm.at[pl.ds(base, B)], srcb.at[b])
        pltpu.sync_copy(dst_hbm.at[pl.ds(base, B)], dstb.at[b])
        pltpu.async_copy(k_hbm.at[srcb.at[b]], krows.at[b], semk[b])
        pltpu.async_copy(q_hbm.at[dstb.at[b]], qrows.at[b], semq[b])

    @pl.loop(0, NCHUNK, step=NB)
    def _grp(g):
        for b in range(NB):
            sb = b & 1
            base = wbase + (g + b) * B
            pltpu.make_async_copy(k_hbm.at[srcb.at[b]], krows.at[b],
                                  semk[b]).wait()
            pltpu.make_async_copy(q_hbm.at[dstb.at[b]], qrows.at[b],
                                  semq[b]).wait()

            def _drain():
                pltpu.make_async_copy(grows2.at[sb], g_hbm.at[pl.ds(0, B)],
                                      semo[sb]).wait()
            if b >= 2:
                _drain()
            else:
                pl.when(g > 0)(_drain)

            @plsc.parallel_loop(0, B)
            def _edge(j):
                for h in range(H):
                    sl = pl.ds(h * D, D)
                    grows2[sb, j, sl] = krows[b, j, sl] * qrows[b, j, sl]

            pltpu.async_copy(grows2.at[sb], g_hbm.at[pl.ds(base, B)], semo[sb])

            @pl.when(g + b + NB < NCHUNK)
            def _pf():
                nb_ = wbase + (g + b + NB) * B
                pltpu.sync_copy(src_hbm.at[pl.ds(nb_, B)], srcb.at[b])
                pltpu.sync_copy(dst_hbm.at[pl.ds(nb_, B)], dstb.at[b])
                pltpu.async_copy(k_hbm.at[srcb.at[b]], krows.at[b], semk[b])
                pltpu.async_copy(q_hbm.at[dstb.at[b]], qrows.at[b], semq[b])

    for sb in range(2):
        pltpu.make_async_copy(grows2.at[sb], g_hbm.at[pl.ds(0, B)],
                              semo[sb]).wait()


def _sc_g(src, dst, k, q):
    mesh = plsc.VectorSubcoreMesh(core_axis_name="c", subcore_axis_name="s")
    fn = pl.kernel(
        _sc_g_body,
        out_type=jax.ShapeDtypeStruct((E, HD), _f32),
        mesh=mesh,
        scratch_types=[
            pltpu.VMEM((NB, B), jnp.int32),
            pltpu.VMEM((NB, B), jnp.int32),
            pltpu.VMEM((NB, B, HD), _f32),
            pltpu.VMEM((NB, B, HD), _f32),
            pltpu.VMEM((2, B, HD), _f32),
        ] + [pltpu.SemaphoreType.DMA] * 12,
    )
    return fn(src, dst, k, q)


def _edge_body(e_ref, we_ref, be_ref, g_ref, eout_ref, s_ref):
    proj = jnp.dot(e_ref[...], we_ref[...], preferred_element_type=_f32) + be_ref[...]
    sc = g_ref[...] * proj
    eout_ref[...] = sc
    r = lax.broadcasted_iota(jnp.int32, (HD, H), 0) // D
    c = lax.broadcasted_iota(jnp.int32, (HD, H), 1)
    m = (r == c).astype(_f32)
    ssum = jnp.dot(sc, m, preferred_element_type=_f32)
    s_ref[...] = jnp.exp(jnp.clip(ssum, -5.0, 5.0))


def _edge_tc(e, we, be, g):
    R = 2000
    grid = E // R
    blk = pl.BlockSpec((R, IN_DIM), lambda i: (i, 0))
    return pl.pallas_call(
        _edge_body,
        grid=(grid,),
        in_specs=[blk,
                  pl.BlockSpec((IN_DIM, HD), lambda i: (0, 0)),
                  pl.BlockSpec((1, HD), lambda i: (0, 0)),
                  blk],
        out_specs=[blk, pl.BlockSpec((R, H), lambda i: (i, 0))],
        out_shape=[jax.ShapeDtypeStruct((E, HD), _f32),
                   jax.ShapeDtypeStruct((E, H), _f32)],
    )(e, we, be.reshape(1, HD), g)


ZGRP = NPAD // 8
ZRPS = ZGRP // NS


def _sc_scatter_body(src_hbm, dst_hbm, v_hbm, s_hbm, ow_hbm, oz_hbm,
                     src_v, dst_v, zdiv_v, mprev, s_v, vrows, zrows,
                     wsh, zsh, sg0, sg1, si0, si1, sw0, sw1):
    cid = lax.axis_index("c")
    sid = lax.axis_index("s")
    wid = sid * NC + cid
    zero16 = jnp.zeros((D,), _f32)
    semg = (sg0, sg1)
    semi = (si0, si1)
    semw = (sw0, sw1)
    wbase = wid * EPW
    iota = lax.iota(jnp.int32, D)
    izero = jnp.zeros((D,), jnp.int32)

    @pl.loop(0, B)
    def _z(r):
        for h in range(H):
            zrows[r, pl.ds(h * D, D)] = zero16

    @pl.loop(0, RPS // B)
    def _zc(t):
        pltpu.sync_copy(zrows, wsh.at[pl.ds(sid * RPS + t * B, B)])

    pltpu.sync_copy(zrows, zsh.at[pl.ds(sid * ZRPS, ZRPS)])

    for c in range(B // D):
        mprev[pl.ds(c * D, D)] = izero
    plsc.subcore_barrier()

    def _fetch(slot, ci, drain_w=True):
        base = wbase + ci * B
        if drain_w:
            pltpu.make_async_copy(vrows.at[slot], wsh.at[dst_v.at[slot]],
                                  semw[slot]).wait()
        pltpu.sync_copy(src_hbm.at[pl.ds(base, B)], src_v.at[slot])
        pltpu.async_copy(v_hbm.at[src_v.at[slot]], vrows.at[slot], semg[slot])
        pltpu.async_copy(dst_hbm.at[pl.ds(base, B)], dst_v.at[slot],
                         semi[slot])
        pltpu.async_copy(s_hbm.at[pl.ds(base * H, B * H)], s_v.at[slot],
                         semi[slot])

    def _consume(slot):
        pltpu.make_async_copy(v_hbm.at[src_v.at[slot]], vrows.at[slot],
                              semg[slot]).wait()
        pltpu.make_async_copy(dst_hbm.at[pl.ds(0, B)], dst_v.at[slot],
                              semi[slot]).wait()
        pltpu.make_async_copy(s_hbm.at[pl.ds(0, B * H)], s_v.at[slot],
                              semi[slot]).wait()

        for c in range(B // D):
            slc = pl.ds(c * D, D)
            zdiv_v[slc] = lax.shift_right_logical(dst_v[slot, slc], 3)

        @plsc.parallel_loop(0, B // D)
        def _grp(c):
            dvec = dst_v[slot, pl.ds(c * D, D)] & 7
            mpv = mprev[pl.ds(c * D, D)]
            for q in range(D // 2):
                sv = s_v[slot, pl.ds((c * 8 + q) * (2 * H), 2 * H)]
                for t in range(2):
                    o = q * 2 + t
                    j = c * D + o
                    for h in range(H):
                        bc = _vtake(sv, jnp.full((D,), t * H + h, jnp.int32))
                        sl = pl.ds(h * D, D)
                        vrows[slot, j, sl] = vrows[slot, j, sl] * bc
                    zg = _vtake(sv, t * H + (iota & 7))
                    zvec = jnp.where(iota < 8, zg, 0.0)
                    zrows[j, pl.ds(mpv[o] * D, D)] = zero16
                    zrows[j, pl.ds(dvec[o] * D, D)] = zvec
            mprev[pl.ds(c * D, D)] = dvec

        pltpu.async_copy(vrows.at[slot], wsh.at[dst_v.at[slot]], semw[slot],
                         add=True)
        pltpu.sync_copy(zrows, zsh.at[zdiv_v], add=True)

    _fetch(0, 0, drain_w=False)
    _fetch(1, 1, drain_w=False)

    @pl.loop(0, NCHUNK // 2)
    def _super(i):
        c0 = i * 2
        _consume(0)
        _fetch(0, c0 + 2)
        _consume(1)

        @pl.when(c0 + 3 < NCHUNK)
        def _pf1():
            _fetch(1, c0 + 3)

    _consume(0)
    for slot in range(2):
        pltpu.make_async_copy(vrows.at[slot], wsh.at[dst_v.at[slot]],
                              semw[slot]).wait()
    plsc.subcore_barrier()

    @pl.loop(0, RPS // B)
    def _out(t):
        row0 = sid * RPS + t * B
        pltpu.sync_copy(wsh.at[pl.ds(row0, B)], vrows.at[0])
        pltpu.sync_copy(vrows.at[0], ow_hbm.at[cid, pl.ds(row0, B)])

    pltpu.sync_copy(zsh.at[pl.ds(sid * ZRPS, ZRPS)], zrows)
    pltpu.sync_copy(zrows, oz_hbm.at[cid, pl.ds(sid * ZRPS, ZRPS)])


def _sc_scatter(src, dst, v, s_flat):
    mesh = plsc.VectorSubcoreMesh(core_axis_name="c", subcore_axis_name="s")
    fn = pl.kernel(
        _sc_scatter_body,
        out_type=(jax.ShapeDtypeStruct((NC, NPAD, HD), _f32),
                  jax.ShapeDtypeStruct((NC, ZGRP, HD), _f32)),
        mesh=mesh,
        scratch_types=[
            pltpu.VMEM((2, B), jnp.int32),
            pltpu.VMEM((2, B), jnp.int32),
            pltpu.VMEM((B,), jnp.int32),
            pltpu.VMEM((B,), jnp.int32),
            pltpu.VMEM((2, B * H), _f32),
            pltpu.VMEM((2, B, HD), _f32),
            pltpu.VMEM((B, HD), _f32),
            pltpu.VMEM_SHARED((NPAD, HD), _f32),
            pltpu.VMEM_SHARED((ZGRP, HD), _f32),
        ] + [pltpu.SemaphoreType.DMA] * 6,
    )
    return fn(src, dst, v, s_flat)


def _combine_body(w_ref, z_ref, o_ref):
    w = w_ref[0] + w_ref[1]
    z = z_ref[0] + z_ref[1]
    j = lax.broadcasted_iota(jnp.int32, (D, HD), 0)
    l = lax.broadcasted_iota(jnp.int32, (D, HD), 1) // D
    m = (j == l).astype(_f32)
    zb = jnp.dot(z, m, preferred_element_type=_f32)
    o_ref[...] = w / (zb + 1e-6)


def _combine(ow, oz):
    R = 2048
    grid = NPAD // R
    return pl.pallas_call(
        _combine_body,
        grid=(grid,),
        in_specs=[pl.BlockSpec((NC, R, HD), lambda i: (0, i, 0)),
                  pl.BlockSpec((NC, R, D), lambda i: (0, i, 0))],
        out_specs=pl.BlockSpec((R, HD), lambda i: (i, 0)),
        out_shape=jax.ShapeDtypeStruct((NPAD, HD), _f32),
    )(ow, oz)


def kernel(h, e, edge_index, W_Q, b_Q, W_K, b_K, W_V, b_V, W_E, b_E):
    src = edge_index[0]
    dst = edge_index[1]
    q, k, v = _proj_qkv(h, W_Q, b_Q, W_K, b_K, W_V, b_V)
    g = _sc_g(src, dst, k, q)
    e_out, s = _edge_tc(e, W_E, b_E, g)
    ow, ozp = _sc_scatter(src, dst, v, s.reshape(E * H))
    h_out = _combine(ow, ozp.reshape(NC, NPAD, D))[:N]
    return h_out.reshape(N, H, D), e_out.reshape(E, H, D)
